# Initial kernel scaffold; baseline (speedup 1.0000x reference)
import jax
import jax.numpy as jnp
from jax.experimental import pallas as pl


def kernel(x, edge_index, W1, b1, W2, b2):
    raise NotImplementedError("write your pallas kernel here")


# speedup vs baseline: 8.3450x; 8.3450x over previous
import functools

import jax
import jax.numpy as jnp
from jax import lax
from jax.experimental import pallas as pl
from jax.experimental.pallas import tpu as pltpu
from jax.experimental.pallas import tpu_sc as plsc

N = 10000
E = 320000
C = 128
NC = 2
NS = 16
NW = NC * NS

NPAD = 10240
RW = NPAD // NS
CHUNK = 128
EPAD = 327680
EW = EPAD // NW
NCHUNK = EW // CHUNK

_sc_mesh = plsc.VectorSubcoreMesh(
    core_axis_name="c", subcore_axis_name="s", num_cores=NC, num_subcores=NS
)


@functools.partial(
    pl.kernel,
    out_type=jax.ShapeDtypeStruct((NC, NPAD, C), jnp.float32),
    mesh=_sc_mesh,
    scratch_types=[
        pltpu.VMEM((CHUNK,), jnp.int32),
        pltpu.VMEM((CHUNK, C), jnp.float32),
        pltpu.VMEM_SHARED((NPAD, C), jnp.float32),
    ],
)
def _sc_degree(dst_hbm, ones_hbm, zrow_hbm, out_hbm, idx_v, ones_v, acc):
    cid = lax.axis_index("c")
    sid = lax.axis_index("s")
    wid = sid * NC + cid
    pltpu.sync_copy(ones_hbm, ones_v)
    pltpu.sync_copy(zrow_hbm, acc.at[pl.ds(sid * RW, RW)])
    plsc.subcore_barrier()
    base = wid * EW

    @pl.loop(0, NCHUNK)
    def _(i):
        pltpu.sync_copy(dst_hbm.at[pl.ds(base + i * CHUNK, CHUNK)], idx_v)
        pltpu.sync_copy(ones_v, acc.at[idx_v], add=True)

    plsc.subcore_barrier()
    pltpu.sync_copy(acc.at[pl.ds(sid * RW, RW)], out_hbm.at[cid, pl.ds(sid * RW, RW)])


@functools.partial(
    pl.kernel,
    out_type=jax.ShapeDtypeStruct((NC, NPAD, C), jnp.float32),
    mesh=_sc_mesh,
    scratch_types=[
        pltpu.VMEM_SHARED((NPAD, C), jnp.float32),
        pltpu.VMEM((CHUNK,), jnp.int32),
        pltpu.VMEM((CHUNK,), jnp.int32),
        pltpu.VMEM((CHUNK, C), jnp.float32),
        pltpu.SemaphoreType.DMA,
    ],
)
def _sc_scatter_rows(u_hbm, src_hbm, dst_hbm, zrows_hbm, out_hbm,
                     acc, sidx, didx, rows, sem):
    cid = lax.axis_index("c")
    sid = lax.axis_index("s")
    wid = sid * NC + cid
    pltpu.sync_copy(zrows_hbm, acc.at[pl.ds(sid * RW, RW)])
    plsc.subcore_barrier()
    base = wid * EW

    @pl.loop(0, NCHUNK)
    def _(i):
        off = base + i * CHUNK
        pltpu.sync_copy(src_hbm.at[pl.ds(off, CHUNK)], sidx)
        pltpu.sync_copy(dst_hbm.at[pl.ds(off, CHUNK)], didx)
        pltpu.async_copy(u_hbm.at[sidx], rows, sem).wait()
        pltpu.sync_copy(rows, acc.at[didx], add=True)

    plsc.subcore_barrier()
    pltpu.sync_copy(acc.at[pl.ds(sid * RW, RW)], out_hbm.at[cid, pl.ds(sid * RW, RW)])


def _dis_from_degp(degp_ref):
    deg = degp_ref[0, : N, 0:1] + degp_ref[1, : N, 0:1] + 1.0
    return lax.rsqrt(deg)


def _tc_scale_in_body(degp_ref, x_ref, u1_ref):
    u1_ref[...] = x_ref[...] * _dis_from_degp(degp_ref)


def _tc_scale_in(degp, x):
    return pl.pallas_call(
        _tc_scale_in_body,
        out_shape=jax.ShapeDtypeStruct((N, C), jnp.float32),
    )(degp, x)


BM = 2000


def _tc_mid_body(degp_ref, p_ref, u1_ref, W1_ref, b1_ref, W2_ref, u2_ref):
    degb = degp_ref[0, :, 0:1] + degp_ref[1, :, 0:1] + 1.0
    dis = lax.rsqrt(degb)
    agg1 = (p_ref[0] + p_ref[1] + u1_ref[...]) * dis
    h1 = jnp.maximum(
        jnp.dot(agg1, W1_ref[...], preferred_element_type=jnp.float32)
        + b1_ref[...],
        0.0,
    )
    u2_ref[...] = jnp.dot(h1, W2_ref[...], preferred_element_type=jnp.float32) * dis


def _tc_mid(degp, part1, u1, W1, b1, W2):
    grid = (N // BM,)
    return pl.pallas_call(
        _tc_mid_body,
        grid=grid,
        in_specs=[
            pl.BlockSpec((2, BM, C), lambda i: (0, i, 0)),
            pl.BlockSpec((2, BM, C), lambda i: (0, i, 0)),
            pl.BlockSpec((BM, C), lambda i: (i, 0)),
            pl.BlockSpec((C, 2 * C), lambda i: (0, 0)),
            pl.BlockSpec((1, 2 * C), lambda i: (0, 0)),
            pl.BlockSpec((2 * C, C), lambda i: (0, 0)),
        ],
        out_specs=pl.BlockSpec((BM, C), lambda i: (i, 0)),
        out_shape=jax.ShapeDtypeStruct((N, C), jnp.float32),
    )(degp, part1, u1, W1, b1, W2)


def _tc_out_body(degp_ref, p_ref, u2_ref, b2_ref, out_ref):
    degb = degp_ref[0, :, 0:1] + degp_ref[1, :, 0:1] + 1.0
    dis = lax.rsqrt(degb)
    out_ref[...] = jnp.maximum(
        (p_ref[0] + p_ref[1] + u2_ref[...]) * dis + b2_ref[...], 0.0
    )


def _tc_out(degp, part2, u2, b2):
    grid = (N // BM,)
    return pl.pallas_call(
        _tc_out_body,
        grid=grid,
        in_specs=[
            pl.BlockSpec((2, BM, C), lambda i: (0, i, 0)),
            pl.BlockSpec((2, BM, C), lambda i: (0, i, 0)),
            pl.BlockSpec((BM, C), lambda i: (i, 0)),
            pl.BlockSpec((1, C), lambda i: (0, 0)),
        ],
        out_specs=pl.BlockSpec((BM, C), lambda i: (i, 0)),
        out_shape=jax.ShapeDtypeStruct((N, C), jnp.float32),
    )(degp, part2, u2, b2)


def kernel(x, edge_index, W1, b1, W2, b2):
    src = edge_index[0].astype(jnp.int32)
    dst = edge_index[1].astype(jnp.int32)
    npad = EPAD - E
    src_p = jnp.concatenate([src, jnp.zeros((npad,), jnp.int32)])
    dst_p = jnp.concatenate(
        [dst, N + (jnp.arange(npad, dtype=jnp.int32) % (NPAD - N - 8))]
    )
    ones_rows = jnp.ones((CHUNK, C), jnp.float32)
    zrows = jnp.zeros((RW, C), jnp.float32)

    degp = _sc_degree(dst_p, ones_rows, zrows)
    u1 = _tc_scale_in(degp, x)
    part1 = _sc_scatter_rows(u1, src_p, dst_p, zrows)
    u2 = _tc_mid(degp, part1, u1, W1, b1.reshape(1, -1), W2)
    part2 = _sc_scatter_rows(u2, src_p, dst_p, zrows)
    out = _tc_out(degp, part2, u2, b2.reshape(1, -1))
    return out


# speedup vs baseline: 10.4894x; 1.2570x over previous
import functools

import jax
import jax.numpy as jnp
from jax import lax
from jax.experimental import pallas as pl
from jax.experimental.pallas import tpu as pltpu
from jax.experimental.pallas import tpu_sc as plsc

N = 10000
E = 320000
C = 128
NC = 2
NS = 16
NW = NC * NS

NPAD = 10240
RW = NPAD // NS
CHUNK = 80
EPAD = 327680
EW = EPAD // NW
NCHUNK = EW // CHUNK
NBUF = 2

_sc_mesh = plsc.VectorSubcoreMesh(
    core_axis_name="c", subcore_axis_name="s", num_cores=NC, num_subcores=NS
)


@functools.partial(
    pl.kernel,
    out_type=jax.ShapeDtypeStruct((NC, NPAD, C), jnp.float32),
    mesh=_sc_mesh,
    scratch_types=[
        pltpu.VMEM((NCHUNK, CHUNK), jnp.int32),
        pltpu.VMEM((CHUNK, C), jnp.float32),
        pltpu.VMEM_SHARED((NPAD, C), jnp.float32),
    ],
)
def _sc_degree(dst_hbm, ones_hbm, zrows_hbm, out_hbm, didx_all, ones_v, acc):
    cid = lax.axis_index("c")
    sid = lax.axis_index("s")
    wid = sid * NC + cid
    pltpu.sync_copy(ones_hbm, ones_v)
    pltpu.sync_copy(dst_hbm.at[wid], didx_all)
    pltpu.sync_copy(zrows_hbm, acc.at[pl.ds(sid * RW, RW)])
    plsc.subcore_barrier()

    @pl.loop(0, NCHUNK)
    def _(i):
        pltpu.sync_copy(ones_v, acc.at[didx_all.at[i]], add=True)

    plsc.subcore_barrier()
    pltpu.sync_copy(acc.at[pl.ds(sid * RW, RW)], out_hbm.at[cid, pl.ds(sid * RW, RW)])


SB = 4


@functools.partial(
    pl.kernel,
    out_type=jax.ShapeDtypeStruct((NC, NPAD, C), jnp.float32),
    mesh=_sc_mesh,
    scratch_types=[
        pltpu.VMEM_SHARED((NPAD, C), jnp.float32),
        pltpu.VMEM((NCHUNK, CHUNK), jnp.int32),
        [pltpu.VMEM((CHUNK,), jnp.int32)] * SB,
        [pltpu.VMEM((CHUNK, C), jnp.float32)] * NBUF,
        [pltpu.SemaphoreType.DMA] * NBUF,
        [pltpu.SemaphoreType.DMA] * SB,
    ],
)
def _sc_scatter_rows(u_hbm, src_hbm, dst_hbm, zrows_hbm, out_hbm,
                     acc, didx_all, sbuf, rows, gsem, ssem):
    cid = lax.axis_index("c")
    sid = lax.axis_index("s")
    wid = sid * NC + cid
    pltpu.sync_copy(dst_hbm.at[wid], didx_all)
    pltpu.sync_copy(zrows_hbm, acc.at[pl.ds(sid * RW, RW)])
    plsc.subcore_barrier()

    for k in range(SB):
        pltpu.async_copy(src_hbm.at[wid, k], sbuf[k], ssem[k])
    for b in range(NBUF):
        pltpu.make_async_copy(src_hbm.at[wid, 0], sbuf[b], ssem[b]).wait()
        pltpu.async_copy(u_hbm.at[sbuf[b]], rows[b], gsem[b])

    @pl.loop(0, NCHUNK // SB)
    def _(o):
        for bb in range(SB):
            i = o * SB + bb
            b = bb % NBUF
            pltpu.make_async_copy(u_hbm.at[sbuf[b]], rows[b], gsem[b]).wait()
            pltpu.sync_copy(rows[b], acc.at[didx_all.at[i]], add=True)
            j = i + NBUF

            @pl.when(j < NCHUNK)
            def _():
                sb = (bb + NBUF) % SB
                pltpu.make_async_copy(src_hbm.at[wid, 0], sbuf[sb], ssem[sb]).wait()
                pltpu.async_copy(u_hbm.at[sbuf[sb]], rows[b], gsem[b])

            j2 = i + SB

            @pl.when(j2 < NCHUNK)
            def _():
                pltpu.async_copy(src_hbm.at[wid, j2], sbuf[bb], ssem[bb])

    plsc.subcore_barrier()
    pltpu.sync_copy(acc.at[pl.ds(sid * RW, RW)], out_hbm.at[cid, pl.ds(sid * RW, RW)])


def _tc_scale_in_body(degp_ref, x_ref, u1_ref):
    deg = degp_ref[0, : N, 0:1] + degp_ref[1, : N, 0:1] + 1.0
    u1_ref[...] = x_ref[...] * lax.rsqrt(deg)


def _tc_scale_in(degp, x):
    return pl.pallas_call(
        _tc_scale_in_body,
        out_shape=jax.ShapeDtypeStruct((N, C), jnp.float32),
    )(degp, x)


BM = 2000


def _tc_mid_body(degp_ref, p_ref, u1_ref, W1_ref, b1_ref, W2_ref, u2_ref):
    degb = degp_ref[0, :, 0:1] + degp_ref[1, :, 0:1] + 1.0
    dis = lax.rsqrt(degb)
    agg1 = (p_ref[0] + p_ref[1] + u1_ref[...]) * dis
    h1 = jnp.maximum(
        jnp.dot(agg1, W1_ref[...], preferred_element_type=jnp.float32)
        + b1_ref[...],
        0.0,
    )
    u2_ref[...] = jnp.dot(h1, W2_ref[...], preferred_element_type=jnp.float32) * dis


def _tc_mid(degp, part1, u1, W1, b1, W2):
    grid = (N // BM,)
    return pl.pallas_call(
        _tc_mid_body,
        grid=grid,
        in_specs=[
            pl.BlockSpec((2, BM, C), lambda i: (0, i, 0)),
            pl.BlockSpec((2, BM, C), lambda i: (0, i, 0)),
            pl.BlockSpec((BM, C), lambda i: (i, 0)),
            pl.BlockSpec((C, 2 * C), lambda i: (0, 0)),
            pl.BlockSpec((1, 2 * C), lambda i: (0, 0)),
            pl.BlockSpec((2 * C, C), lambda i: (0, 0)),
        ],
        out_specs=pl.BlockSpec((BM, C), lambda i: (i, 0)),
        out_shape=jax.ShapeDtypeStruct((N, C), jnp.float32),
    )(degp, part1, u1, W1, b1, W2)


def _tc_out_body(degp_ref, p_ref, u2_ref, b2_ref, out_ref):
    degb = degp_ref[0, :, 0:1] + degp_ref[1, :, 0:1] + 1.0
    dis = lax.rsqrt(degb)
    out_ref[...] = jnp.maximum(
        (p_ref[0] + p_ref[1] + u2_ref[...]) * dis + b2_ref[...], 0.0
    )


def _tc_out(degp, part2, u2, b2):
    grid = (N // BM,)
    return pl.pallas_call(
        _tc_out_body,
        grid=grid,
        in_specs=[
            pl.BlockSpec((2, BM, C), lambda i: (0, i, 0)),
            pl.BlockSpec((2, BM, C), lambda i: (0, i, 0)),
            pl.BlockSpec((BM, C), lambda i: (i, 0)),
            pl.BlockSpec((1, C), lambda i: (0, 0)),
        ],
        out_specs=pl.BlockSpec((BM, C), lambda i: (i, 0)),
        out_shape=jax.ShapeDtypeStruct((N, C), jnp.float32),
    )(degp, part2, u2, b2)


def kernel(x, edge_index, W1, b1, W2, b2):
    src = edge_index[0].astype(jnp.int32)
    dst = edge_index[1].astype(jnp.int32)
    npad = EPAD - E
    src_p = jnp.concatenate([src, jnp.zeros((npad,), jnp.int32)])
    dst_p = jnp.concatenate(
        [dst, N + (jnp.arange(npad, dtype=jnp.int32) % (NPAD - N - 8))]
    )
    src3 = src_p.reshape(NW, NCHUNK, CHUNK)
    dst3 = dst_p.reshape(NW, NCHUNK, CHUNK)
    ones_rows = jnp.ones((CHUNK, C), jnp.float32)
    zrows = jnp.zeros((RW, C), jnp.float32)

    degp = _sc_degree(dst3, ones_rows, zrows)
    u1 = _tc_scale_in(degp, x)
    part1 = _sc_scatter_rows(u1, src3, dst3, zrows)
    u2 = _tc_mid(degp, part1, u1, W1, b1.reshape(1, -1), W2)
    part2 = _sc_scatter_rows(u2, src3, dst3, zrows)
    out = _tc_out(degp, part2, u2, b2.reshape(1, -1))
    return out
